# baseline (speedup 1.0000x reference)
import jax
import jax.numpy as jnp
from jax.experimental import pallas as pl
from jax.experimental.pallas import tpu as pltpu


def _mm(a, b):
    return jax.lax.dot_general(a, b, (((a.ndim - 1,), (0,)), ((), ())),
                               preferred_element_type=jnp.float32)


def _stencil_packed(v, Ch):
    ip = jnp.concatenate([v[1:], v[:1]], axis=0)
    im = jnp.concatenate([v[-1:], v[:-1]], axis=0)
    swap = jnp.concatenate([v[:, :, Ch:], v[:, :, :Ch]], axis=2)
    swap_jp = jnp.concatenate([swap[:, 1:], swap[:, :1]], axis=1)
    swap_jm = jnp.concatenate([swap[:, -1:], swap[:, :-1]], axis=1)
    lane = jax.lax.broadcasted_iota(jnp.int32, v.shape, 2)
    jp = jnp.where(lane < Ch, swap, swap_jp)
    jm = jnp.where(lane < Ch, swap_jm, swap)
    return (ip + im) + (jp + jm)


def _tile_body(x1_ref, x2_ref, wup_ref, a1_ref, b1v_ref, bias1_ref,
               a2_ref, bias2_ref, out_ref):
    H = x1_ref.shape[1]
    C = x1_ref.shape[3]
    n = 2 * H
    P = x2_ref.shape[3]
    Ch = P // 2

    x1 = x1_ref[0].reshape(H * H, C)
    x2 = x2_ref[0]

    B = _mm(x1, wup_ref[...])
    b0 = B[:, :P].reshape(H, H, P)
    b1 = B[:, P:].reshape(H, H, P)
    up = jnp.stack([b0, b1], axis=1).reshape(n, H, P)

    M = (_mm(x2.reshape(n * H, P), a1_ref[...])
         + _mm(up.reshape(n * H, P), b1v_ref[...])).reshape(n, H, 2 * P)
    h1 = jax.nn.relu(M[:, :, :P] + _stencil_packed(M[:, :, P:], Ch) * 0.25
                     + bias1_ref[...].reshape(1, 1, P))

    M2 = _mm(h1.reshape(n * H, P), a2_ref[...]).reshape(n, H, 2 * P)
    out_ref[0] = jax.nn.relu(M2[:, :, :P]
                             + _stencil_packed(M2[:, :, P:], Ch) * 0.25
                             + bias2_ref[...].reshape(1, 1, P))


def kernel(x1, x2, W_up, b_up, W_self1, W_neigh1, b1, W_self2, W_neigh2, b2):
    B, T, H, Wd, C = x1.shape
    n = 2 * H
    Ch = x2.shape[-1]
    P = 2 * Ch
    G = B * T
    f32 = jnp.float32

    x1r = x1.reshape(G, H, Wd, C)
    x2n = x2.reshape(G, n, n, Ch)

    wup = jnp.concatenate([W_up[:, :, 0, 0], W_up[:, :, 0, 1],
                           W_up[:, :, 1, 0], W_up[:, :, 1, 1]], axis=1)

    def blockdiag(W):
        Z = jnp.zeros_like(W)
        return jnp.concatenate([jnp.concatenate([W, Z], axis=1),
                                jnp.concatenate([Z, W], axis=1)], axis=0)

    A1 = jnp.concatenate([blockdiag(W_self1[:Ch]), blockdiag(W_neigh1[:Ch])],
                         axis=1)
    B1 = jnp.concatenate([blockdiag(W_self1[Ch:]), blockdiag(W_neigh1[Ch:])],
                         axis=1)
    A2 = jnp.concatenate([blockdiag(W_self2), blockdiag(W_neigh2)], axis=1)

    bb1 = b1 + b_up @ W_self1[Ch:] + b_up @ W_neigh1[Ch:]
    b1_p = jnp.tile(bb1, 2).reshape(1, P)
    b2_p = jnp.tile(b2, 2).reshape(1, P)

    full = lambda shp: pl.BlockSpec(shp, lambda g: (0,) * len(shp))

    def run(x1h, x2h):
        Gh = x1h.shape[0]
        return pl.pallas_call(
            _tile_body,
            grid=(Gh,),
            in_specs=[
                pl.BlockSpec((1, H, Wd, C), lambda g: (g, 0, 0, 0)),
                pl.BlockSpec((1, n, H, P), lambda g: (g, 0, 0, 0)),
                full((C, 2 * P)),
                full((P, 2 * P)),
                full((P, 2 * P)),
                full((1, P)),
                full((P, 2 * P)),
                full((1, P)),
            ],
            out_specs=pl.BlockSpec((1, n, H, P), lambda g: (g, 0, 0, 0)),
            out_shape=jax.ShapeDtypeStruct((Gh, n, H, P), f32),
        )(x1h, x2h, wup, A1, B1, b1_p, A2, b2_p)

    Gh = G // 2
    if Gh == 0:
        out = run(x1r, x2n.reshape(G, n, H, P))
        return out.reshape(B, T, n, n, Ch)
    o0 = run(x1r[:Gh], x2n[:Gh].reshape(Gh, n, H, P))
    o1 = run(x1r[Gh:], x2n[Gh:].reshape(G - Gh, n, H, P))
    o0n = o0.reshape(1, Gh, n, n, Ch)
    o1n = o1.reshape(1, G - Gh, n, n, Ch)
    return jnp.concatenate([o0n, o1n], axis=1).reshape(B, T, n, n, Ch)


# speedup vs baseline: 1.3911x; 1.3911x over previous
import jax
import jax.numpy as jnp
from jax.experimental import pallas as pl
from jax.experimental.pallas import tpu as pltpu


def _mm(a, b):
    return jax.lax.dot_general(a, b, (((a.ndim - 1,), (0,)), ((), ())),
                               preferred_element_type=jnp.float32)


def _stencil_packed(v, Ch):
    ip = jnp.concatenate([v[1:], v[:1]], axis=0)
    im = jnp.concatenate([v[-1:], v[:-1]], axis=0)
    swap = jnp.concatenate([v[:, :, Ch:], v[:, :, :Ch]], axis=2)
    swap_jp = jnp.concatenate([swap[:, 1:], swap[:, :1]], axis=1)
    swap_jm = jnp.concatenate([swap[:, -1:], swap[:, :-1]], axis=1)
    lane = jax.lax.broadcasted_iota(jnp.int32, v.shape, 2)
    jp = jnp.where(lane < Ch, swap, swap_jp)
    jm = jnp.where(lane < Ch, swap_jm, swap)
    return (ip + im) + (jp + jm)


def _tile_body(x1_ref, x2_ref, wup_ref, a1_ref, b1v_ref, bias1_ref,
               a2_ref, bias2_ref, out_ref):
    H = x1_ref.shape[1]
    C = x1_ref.shape[3]
    n = 2 * H
    P = x2_ref.shape[3]
    Ch = P // 2

    x1 = x1_ref[0].reshape(H * H, C)
    x2 = x2_ref[0]

    B = _mm(x1, wup_ref[...])
    b0 = B[:, :P].reshape(H, H, P)
    b1 = B[:, P:].reshape(H, H, P)
    up = jnp.stack([b0, b1], axis=1).reshape(n, H, P)

    M = (_mm(x2.reshape(n * H, P), a1_ref[...])
         + _mm(up.reshape(n * H, P), b1v_ref[...])).reshape(n, H, 2 * P)
    h1 = jax.nn.relu(M[:, :, :P] + _stencil_packed(M[:, :, P:], Ch) * 0.25
                     + bias1_ref[...].reshape(1, 1, P))

    M2 = _mm(h1.reshape(n * H, P), a2_ref[...]).reshape(n, H, 2 * P)
    out_ref[0] = jax.nn.relu(M2[:, :, :P]
                             + _stencil_packed(M2[:, :, P:], Ch) * 0.25
                             + bias2_ref[...].reshape(1, 1, P))


def kernel(x1, x2, W_up, b_up, W_self1, W_neigh1, b1, W_self2, W_neigh2, b2):
    B, T, H, Wd, C = x1.shape
    n = 2 * H
    Ch = x2.shape[-1]
    P = 2 * Ch
    G = B * T
    f32 = jnp.float32

    x1r = x1.reshape(G, H, Wd, C)
    x2r = x2.reshape(G, n, H, P)

    wup = jnp.concatenate([W_up[:, :, 0, 0], W_up[:, :, 0, 1],
                           W_up[:, :, 1, 0], W_up[:, :, 1, 1]], axis=1)

    def blockdiag(W):
        Z = jnp.zeros_like(W)
        return jnp.concatenate([jnp.concatenate([W, Z], axis=1),
                                jnp.concatenate([Z, W], axis=1)], axis=0)

    A1 = jnp.concatenate([blockdiag(W_self1[:Ch]), blockdiag(W_neigh1[:Ch])],
                         axis=1)
    B1 = jnp.concatenate([blockdiag(W_self1[Ch:]), blockdiag(W_neigh1[Ch:])],
                         axis=1)
    A2 = jnp.concatenate([blockdiag(W_self2), blockdiag(W_neigh2)], axis=1)

    bb1 = b1 + b_up @ W_self1[Ch:] + b_up @ W_neigh1[Ch:]
    b1_p = jnp.tile(bb1, 2).reshape(1, P)
    b2_p = jnp.tile(b2, 2).reshape(1, P)

    full = lambda shp: pl.BlockSpec(shp, lambda g: (0,) * len(shp))
    out = pl.pallas_call(
        _tile_body,
        grid=(G,),
        in_specs=[
            pl.BlockSpec((1, H, Wd, C), lambda g: (g, 0, 0, 0)),
            pl.BlockSpec((1, n, H, P), lambda g: (g, 0, 0, 0)),
            full((C, 2 * P)),
            full((P, 2 * P)),
            full((P, 2 * P)),
            full((1, P)),
            full((P, 2 * P)),
            full((1, P)),
        ],
        out_specs=pl.BlockSpec((1, n, H, P), lambda g: (g, 0, 0, 0)),
        out_shape=jax.ShapeDtypeStruct((G, n, H, P), f32),
    )(x1r, x2r, wup, A1, B1, b1_p, A2, b2_p)
    return out.reshape(B, T, n, n, Ch)
